# baseline (speedup 1.0000x reference)
import jax
import jax.numpy as jnp
from jax.experimental import pallas as pl
from jax.experimental.pallas import tpu as pltpu

_B = 16384
_C = 1000
_BLK = 512
_NBLK = _B // _BLK


def _loss_body(lab_ref, pred_ref, out_ref, acc_ref):
    i = pl.program_id(0)

    @pl.when(i == 0)
    def _():
        acc_ref[0, 0] = 0.0

    x = pred_ref[...]
    labi = lab_ref[0].reshape(_BLK, 1)
    cols = jax.lax.broadcasted_iota(jnp.int32, (_BLK, _C), 1)
    sel = jnp.sum(jnp.where(cols == labi, x, 0.0), axis=1, keepdims=True)
    acc_ref[0, 0] += jnp.sum(jnp.log(sel + 1e-8))

    @pl.when(i == _NBLK - 1)
    def _():
        out_ref[0, 0] = acc_ref[0, 0] * (-1.0 / (_B * _C))


def kernel(pred, label):
    lab3 = label.astype(jnp.int32).reshape(_NBLK, 1, _BLK)
    out = pl.pallas_call(
        _loss_body,
        grid=(_NBLK,),
        in_specs=[
            pl.BlockSpec((1, 1, _BLK), lambda i: (i, 0, 0)),
            pl.BlockSpec((_BLK, _C), lambda i: (i, 0)),
        ],
        out_specs=pl.BlockSpec(memory_space=pltpu.SMEM),
        out_shape=jax.ShapeDtypeStruct((1, 1), jnp.float32),
        scratch_shapes=[pltpu.SMEM((1, 1), jnp.float32)],
    )(lab3, pred)
    return out[0, 0]


# speedup vs baseline: 1.1070x; 1.1070x over previous
import jax
import jax.numpy as jnp
from jax.experimental import pallas as pl
from jax.experimental.pallas import tpu as pltpu

_B = 16384
_C = 1000
_BLK = 2048
_NBLK = _B // _BLK


def _loss_body(lab_ref, pred_ref, out_ref, acc_ref):
    i = pl.program_id(0)

    @pl.when(i == 0)
    def _():
        acc_ref[0, 0] = 0.0

    x = pred_ref[...]
    labi = lab_ref[0].reshape(_BLK, 1)
    cols = jax.lax.broadcasted_iota(jnp.int32, (_BLK, _C), 1)
    sel = jnp.sum(jnp.where(cols == labi, x, 0.0), axis=1, keepdims=True)
    acc_ref[0, 0] += jnp.sum(jnp.log(sel + 1e-8))

    @pl.when(i == _NBLK - 1)
    def _():
        out_ref[0, 0] = acc_ref[0, 0] * (-1.0 / (_B * _C))


def kernel(pred, label):
    lab3 = label.astype(jnp.int32).reshape(_NBLK, 1, _BLK)
    out = pl.pallas_call(
        _loss_body,
        grid=(_NBLK,),
        in_specs=[
            pl.BlockSpec((1, 1, _BLK), lambda i: (i, 0, 0)),
            pl.BlockSpec((_BLK, _C), lambda i: (i, 0)),
        ],
        out_specs=pl.BlockSpec(memory_space=pltpu.SMEM),
        out_shape=jax.ShapeDtypeStruct((1, 1), jnp.float32),
        scratch_shapes=[pltpu.SMEM((1, 1), jnp.float32)],
    )(lab3, pred)
    return out[0, 0]
